# Initial kernel scaffold; baseline (speedup 1.0000x reference)
import jax
import jax.numpy as jnp
from jax.experimental import pallas as pl


def kernel(sequence, atom_mapping, segment_label, token_table, segment_table, atom_table):
    raise NotImplementedError("write your pallas kernel here")


# speedup vs baseline: 2.0556x; 2.0556x over previous
import functools

import numpy as np
import jax
import jax.numpy as jnp
from jax import lax
from jax.experimental import pallas as pl
from jax.experimental.pallas import tpu as pltpu
from jax.experimental.pallas import tpu_sc as plsc

S = 512
B = 256
D = 512
N = S * B

_info = plsc.get_sparse_core_info()
NC, NS, L = _info.num_cores, _info.num_subcores, _info.num_lanes
NW = NC * NS
PER_W = N // NW
C = 64
NCHUNK = PER_W // C
ROWS_PER_S = B // C
S_PER_W = PER_W // B


def _positional_pe_np():
    position = np.arange(S, dtype=np.float32)[:, None]
    div_term = np.exp(
        np.arange(0, D, 2, dtype=np.float32) * -(np.log(10000.0) / D))
    pe = np.zeros((S, D), dtype=np.float32)
    pe[:, 0::2] = np.sin(position * div_term)
    pe[:, 1::2] = np.cos(position * div_term)
    return pe


_PE = _positional_pe_np()

_mesh = plsc.VectorSubcoreMesh(core_axis_name="c", subcore_axis_name="s")


@functools.partial(
    pl.kernel,
    mesh=_mesh,
    out_type=jax.ShapeDtypeStruct((N, D), jnp.float32),
    scratch_types=[
        pltpu.VMEM((NCHUNK, C), jnp.int32),
        pltpu.VMEM((NCHUNK, C), jnp.int32),
        pltpu.VMEM((NCHUNK, C), jnp.int32),
        pltpu.VMEM((C, D), jnp.float32),
        pltpu.VMEM((C, D), jnp.float32),
        pltpu.VMEM((C, D), jnp.float32),
        pltpu.SemaphoreType.DMA,
        pltpu.SemaphoreType.DMA,
        pltpu.SemaphoreType.DMA,
    ],
)
def _emb_kernel(seq_hbm, atom_hbm, seg_hbm, tok_tab, atom_tab, comb_tab,
                out_hbm, tok_i, atom_i, comb_i, tok_r, atom_r, comb_r,
                sem_t, sem_a, sem_c):
    wid = lax.axis_index("s") * NC + lax.axis_index("c")
    base_w = wid * PER_W

    pltpu.sync_copy(seq_hbm.at[wid], tok_i)
    pltpu.sync_copy(atom_hbm.at[wid], atom_i)
    pltpu.sync_copy(seg_hbm.at[wid], comb_i)

    def idx_body(j, _):
        s3 = 3 * (wid * S_PER_W + j // ROWS_PER_S)
        for t in range(C // L):
            sl = pl.ds(t * L, L)
            comb_i[j, sl] = comb_i[j, sl] + s3
        return 0

    lax.fori_loop(0, NCHUNK, idx_body, 0, unroll=False)

    def chunk_body(j, _):
        cp_t = pltpu.async_copy(tok_tab.at[tok_i.at[j]], tok_r, sem_t)
        cp_a = pltpu.async_copy(atom_tab.at[atom_i.at[j]], atom_r, sem_a)
        cp_c = pltpu.async_copy(comb_tab.at[comb_i.at[j]], comb_r, sem_c)
        cp_t.wait()
        cp_a.wait()
        cp_c.wait()

        def row_body(i, _):
            for t in range(D // L):
                sl = pl.ds(t * L, L)
                tok_r[i, sl] = tok_r[i, sl] + atom_r[i, sl] + comb_r[i, sl]
            return 0

        lax.fori_loop(0, C, row_body, 0, unroll=False)

        pltpu.sync_copy(tok_r, out_hbm.at[pl.ds(base_w + j * C, C)])
        return 0

    lax.fori_loop(0, NCHUNK, chunk_body, 0, unroll=False)


def kernel(sequence, atom_mapping, segment_label, token_table, segment_table,
           atom_table):
    seq = sequence.reshape(NW, NCHUNK, C).astype(jnp.int32)
    atom = atom_mapping.reshape(NW, NCHUNK, C).astype(jnp.int32)
    seg = segment_label.reshape(NW, NCHUNK, C).astype(jnp.int32)
    comb = (jnp.asarray(_PE)[:, None, :] + segment_table[None, :, :])
    comb = comb.reshape(S * 3, D)
    out = _emb_kernel(seq, atom, seg, token_table, atom_table, comb)
    return out.reshape(S, B, D)
